# baseline (speedup 1.0000x reference)
import jax
import jax.numpy as jnp
from jax.experimental import pallas as pl

_B = 512
_S = 1024
_R1 = 8192
_R2 = 24576
_NS = 256
_NM = 256
_NSMT = 4096
_LF = 1e-6
_NAL = 2.0
_EPS = 1e-30

_C1 = 1024
_C2 = 1024
_HI = (_R1 + _R2) // 128


def _sigmoid(x):
    return 1.0 / (1.0 + jnp.exp(-x))


def _med(t_col):
    Tg = 10.0 + 290.0 * _sigmoid(1e-3 * t_col)
    return jnp.log(Tg / 300.0), 1.0 / Tg


def _p1st_kernel(t_ref, y_ref, a_ref, b_ref, g_ref, r11_ref, p1_ref,
                 surf_ref, mant_ref, smtr_ref, smtc_ref,
                 ra_ref, net_ref, ys_ref, ym_ref, cnt_ref, kmat_ref):
    i = pl.program_id(0)

    @pl.when(i == 0)
    def _init():
        iota_s = jax.lax.broadcasted_iota(jnp.int32, (_S, _NS), 0)
        cnt = jnp.sum((iota_s == surf_ref[...]).astype(jnp.float32),
                      axis=1, keepdims=True)
        cntm = jnp.sum((iota_s == mant_ref[...]).astype(jnp.float32),
                       axis=1, keepdims=True)
        cnt_ref[...] = cnt
        ys_ref[...] = jnp.dot(y_ref[...], cnt,
                              preferred_element_type=jnp.float32)
        ym_ref[...] = jnp.dot(y_ref[...], cntm,
                              preferred_element_type=jnp.float32)
        hi_row = smtr_ref[...] // 128
        lo_col = smtc_ref[...] % 128
        mh = (jax.lax.broadcasted_iota(jnp.int32, (_HI, _NSMT), 0)
              == hi_row).astype(jnp.float32)
        ml = (jax.lax.broadcasted_iota(jnp.int32, (_NSMT, 128), 1)
              == lo_col).astype(jnp.float32)
        kmat_ref[...] = jnp.dot(mh, ml, preferred_element_type=jnp.float32)
        net_ref[...] = jnp.zeros_like(net_ref)

    L, invT = _med(t_ref[...])
    c = a_ref[...] * jnp.exp(b_ref[...] * L - g_ref[...] * invT)
    iota_sub = jax.lax.broadcasted_iota(jnp.int32, (_S, _C1), 0)
    G = (iota_sub == r11_ref[...]).astype(jnp.float32)
    P = (iota_sub == p1_ref[...]).astype(jnp.float32)
    yA = jnp.dot(y_ref[...], G, preferred_element_type=jnp.float32)
    ra = c * yA
    ra_ref[...] = ra
    w = jnp.sum((P - G) * cnt_ref[...], axis=0, keepdims=True)
    net_ref[...] += jnp.sum(ra * w, axis=1, keepdims=True)


def _p2nd_kernel(t_ref, y_ref, a_ref, b_ref, g_ref, r12_ref, r22_ref, p2_ref,
                 cnt_ref, rb_ref, net_ref):
    i = pl.program_id(0)

    @pl.when(i == 0)
    def _init():
        net_ref[...] = jnp.zeros_like(net_ref)

    t = t_ref[...]
    L, invT = _med(t)
    den = jnp.exp(4.0 + 2.0 * jnp.tanh(5e-4 * t))
    c = a_ref[...] * jnp.exp(b_ref[...] * L - g_ref[...] * invT)
    iota_sub = jax.lax.broadcasted_iota(jnp.int32, (_S, _C2), 0)
    Ga = (iota_sub == r12_ref[...]).astype(jnp.float32)
    Gb = (iota_sub == r22_ref[...]).astype(jnp.float32)
    P = (iota_sub == p2_ref[...]).astype(jnp.float32)
    yB1 = jnp.dot(y_ref[...], Ga, preferred_element_type=jnp.float32)
    yB2 = jnp.dot(y_ref[...], Gb, preferred_element_type=jnp.float32)
    rb = c * yB1 * yB2 * den
    rb_ref[...] = rb
    w = jnp.sum((P - Ga - Gb) * cnt_ref[...], axis=0, keepdims=True)
    net_ref[...] += jnp.sum(rb * w, axis=1, keepdims=True)


def _scale(net1, net2, ys, ym):
    nl = _LF * (ys + ym)
    decay = jnp.minimum(_NAL / (nl + _EPS), 1.0)
    sc = decay * _sigmoid(net1 + net2)
    return jnp.log(sc)


def _s1st_kernel(ra_ref, p1_ref, r11_ref, k_ref, n1_ref, n2_ref,
                 ys_ref, ym_ref, dy_ref):
    i = pl.program_id(0)

    @pl.when(i == 0)
    def _init():
        dy_ref[...] = jnp.zeros_like(dy_ref)

    lsc = _scale(n1_ref[...], n2_ref[...], ys_ref[...], ym_ref[...])
    S1 = jnp.exp(k_ref[...] * lsc)
    rs = ra_ref[...] * S1
    iota_lane = jax.lax.broadcasted_iota(jnp.int32, (_C1, _S), 1)
    M = ((iota_lane == p1_ref[...]).astype(jnp.float32)
         - (iota_lane == r11_ref[...]).astype(jnp.float32))
    dy_ref[...] += jnp.dot(rs, M, preferred_element_type=jnp.float32)


def _s2nd_kernel(rb_ref, p2_ref, r12_ref, r22_ref, k_ref, n1_ref, n2_ref,
                 ys_ref, ym_ref, dy1_ref, dy_ref):
    i = pl.program_id(0)

    @pl.when(i == 0)
    def _init():
        dy_ref[...] = dy1_ref[...]

    lsc = _scale(n1_ref[...], n2_ref[...], ys_ref[...], ym_ref[...])
    S2 = jnp.exp(k_ref[...] * lsc)
    rs = rb_ref[...] * S2
    iota_lane = jax.lax.broadcasted_iota(jnp.int32, (_C2, _S), 1)
    M = ((iota_lane == p2_ref[...]).astype(jnp.float32)
         - (iota_lane == r12_ref[...]).astype(jnp.float32)
         - (iota_lane == r22_ref[...]).astype(jnp.float32))
    dy_ref[...] += jnp.dot(rs, M, preferred_element_type=jnp.float32)


def _row(x, n):
    return x.astype(jnp.int32).reshape(1, n)


def _col(x, n):
    return x.astype(jnp.int32).reshape(n, 1)


def kernel(t_in, y_in, alpha_1st, beta_1st, gamma_1st, alpha_2nd, beta_2nd,
           gamma_2nd, r1_1st, p_1st, r1_2nd, r2_2nd, p_2nd,
           inds_surf, inds_mant, inds_smt):
    f32 = jnp.float32
    t_col = t_in.astype(f32).reshape(_B, 1)
    y = y_in.astype(f32)
    a1 = alpha_1st.astype(f32).reshape(1, _R1)
    b1 = beta_1st.astype(f32).reshape(1, _R1)
    g1 = gamma_1st.astype(f32).reshape(1, _R1)
    a2 = alpha_2nd.astype(f32).reshape(1, _R2)
    b2 = beta_2nd.astype(f32).reshape(1, _R2)
    g2 = gamma_2nd.astype(f32).reshape(1, _R2)

    const = lambda *bs: pl.BlockSpec(bs, lambda i: (0,) * len(bs))
    rowblk = lambda c: pl.BlockSpec((1, c), lambda i: (0, i))
    colblk = lambda c: pl.BlockSpec((c, 1), lambda i: (i, 0))

    n1 = _R1 // _C1
    ra, net1, ysurf, ymant, cnt, kmat = pl.pallas_call(
        _p1st_kernel,
        grid=(n1,),
        in_specs=[
            const(_B, 1), const(_B, _S),
            rowblk(_C1), rowblk(_C1), rowblk(_C1),
            rowblk(_C1), rowblk(_C1),
            const(1, _NS), const(1, _NM),
            const(1, _NSMT), const(_NSMT, 1),
        ],
        out_specs=[
            pl.BlockSpec((_B, _C1), lambda i: (0, i)),
            const(_B, 1), const(_B, 1), const(_B, 1),
            const(_S, 1), const(_HI, 128),
        ],
        out_shape=[
            jax.ShapeDtypeStruct((_B, _R1), f32),
            jax.ShapeDtypeStruct((_B, 1), f32),
            jax.ShapeDtypeStruct((_B, 1), f32),
            jax.ShapeDtypeStruct((_B, 1), f32),
            jax.ShapeDtypeStruct((_S, 1), f32),
            jax.ShapeDtypeStruct((_HI, 128), f32),
        ],
    )(t_col, y, a1, b1, g1, _row(r1_1st, _R1), _row(p_1st, _R1),
      _row(inds_surf, _NS), _row(inds_mant, _NM),
      _row(inds_smt, _NSMT), _col(inds_smt, _NSMT))

    n2 = _R2 // _C2
    rb, net2 = pl.pallas_call(
        _p2nd_kernel,
        grid=(n2,),
        in_specs=[
            const(_B, 1), const(_B, _S),
            rowblk(_C2), rowblk(_C2), rowblk(_C2),
            rowblk(_C2), rowblk(_C2), rowblk(_C2),
            const(_S, 1),
        ],
        out_specs=[
            pl.BlockSpec((_B, _C2), lambda i: (0, i)),
            const(_B, 1),
        ],
        out_shape=[
            jax.ShapeDtypeStruct((_B, _R2), f32),
            jax.ShapeDtypeStruct((_B, 1), f32),
        ],
    )(t_col, y, a2, b2, g2, _row(r1_2nd, _R2), _row(r2_2nd, _R2),
      _row(p_2nd, _R2), cnt)

    k_row = kmat.reshape(1, _R1 + _R2)
    k1 = k_row[:, :_R1]
    k2 = k_row[:, _R1:]

    dy1 = pl.pallas_call(
        _s1st_kernel,
        grid=(n1,),
        in_specs=[
            pl.BlockSpec((_B, _C1), lambda i: (0, i)),
            colblk(_C1), colblk(_C1), rowblk(_C1),
            const(_B, 1), const(_B, 1), const(_B, 1), const(_B, 1),
        ],
        out_specs=const(_B, _S),
        out_shape=jax.ShapeDtypeStruct((_B, _S), f32),
    )(ra, _col(p_1st, _R1), _col(r1_1st, _R1), k1, net1, net2, ysurf, ymant)

    dy = pl.pallas_call(
        _s2nd_kernel,
        grid=(n2,),
        in_specs=[
            pl.BlockSpec((_B, _C2), lambda i: (0, i)),
            colblk(_C2), colblk(_C2), colblk(_C2), rowblk(_C2),
            const(_B, 1), const(_B, 1), const(_B, 1), const(_B, 1),
            const(_B, _S),
        ],
        out_specs=const(_B, _S),
        out_shape=jax.ShapeDtypeStruct((_B, _S), f32),
    )(rb, _col(p_2nd, _R2), _col(r1_2nd, _R2), _col(r2_2nd, _R2), k2,
      net1, net2, ysurf, ymant, dy1)

    return dy


# speedup vs baseline: 1.0010x; 1.0010x over previous
import jax
import jax.numpy as jnp
from jax.experimental import pallas as pl

_B = 512
_S = 1024
_R1 = 8192
_R2 = 24576
_NS = 256
_NM = 256
_NSMT = 4096
_LF = 1e-6
_NAL = 2.0
_EPS = 1e-30

_C1 = 1024
_C2 = 1024
_HI = (_R1 + _R2) // 128

_INTERPRET = False


def _sigmoid(x):
    return 1.0 / (1.0 + jnp.exp(-x))


def _med(t_col):
    Tg = 10.0 + 290.0 * _sigmoid(1e-3 * t_col)
    return jnp.log(Tg / 300.0), 1.0 / Tg


def _p1st_kernel(t_ref, y_ref, a_ref, b_ref, g_ref, r11_ref, p1_ref,
                 surf_ref, mant_ref, smtr_ref, smtc_ref,
                 ra_ref, net_ref, ys_ref, ym_ref, cnt_ref, kmat_ref,
                 dyp_ref):
    i = pl.program_id(0)

    @pl.when(i == 0)
    def _init():
        iota_s = jax.lax.broadcasted_iota(jnp.int32, (_S, _NS), 0)
        cnt = jnp.sum((iota_s == surf_ref[...]).astype(jnp.float32),
                      axis=1, keepdims=True)
        cntm = jnp.sum((iota_s == mant_ref[...]).astype(jnp.float32),
                       axis=1, keepdims=True)
        cnt_ref[...] = cnt
        ys_ref[...] = jnp.dot(y_ref[...], cnt,
                              preferred_element_type=jnp.float32)
        ym_ref[...] = jnp.dot(y_ref[...], cntm,
                              preferred_element_type=jnp.float32)
        hi_row = smtr_ref[...] // 128
        lo_col = smtc_ref[...] % 128
        mh = (jax.lax.broadcasted_iota(jnp.int32, (_HI, _NSMT), 0)
              == hi_row).astype(jnp.float32)
        ml = (jax.lax.broadcasted_iota(jnp.int32, (_NSMT, 128), 1)
              == lo_col).astype(jnp.float32)
        kmat_ref[...] = jnp.dot(mh, ml, preferred_element_type=jnp.float32)
        net_ref[...] = jnp.zeros_like(net_ref)
        dyp_ref[...] = jnp.zeros_like(dyp_ref)

    L, invT = _med(t_ref[...])
    c = a_ref[...] * jnp.exp(b_ref[...] * L - g_ref[...] * invT)
    iota_sub = jax.lax.broadcasted_iota(jnp.int32, (_S, _C1), 0)
    G = (iota_sub == r11_ref[...]).astype(jnp.float32)
    P = (iota_sub == p1_ref[...]).astype(jnp.float32)
    yA = jnp.dot(y_ref[...], G, preferred_element_type=jnp.float32)
    ra = c * yA
    ra_ref[...] = ra
    w = jnp.sum((P - G) * cnt_ref[...], axis=0, keepdims=True)
    net_ref[...] += jnp.sum(ra * w, axis=1, keepdims=True)
    dyp_ref[...] += jax.lax.dot_general(
        ra, P - G, (((1,), (1,)), ((), ())),
        preferred_element_type=jnp.float32)


def _p2nd_kernel(t_ref, y_ref, a_ref, b_ref, g_ref, r12_ref, r22_ref, p2_ref,
                 cnt_ref, rb_ref, net_ref):
    i = pl.program_id(0)

    @pl.when(i == 0)
    def _init():
        net_ref[...] = jnp.zeros_like(net_ref)

    t = t_ref[...]
    L, invT = _med(t)
    den = jnp.exp(4.0 + 2.0 * jnp.tanh(5e-4 * t))
    c = a_ref[...] * jnp.exp(b_ref[...] * L - g_ref[...] * invT)
    iota_sub = jax.lax.broadcasted_iota(jnp.int32, (_S, _C2), 0)
    Ga = (iota_sub == r12_ref[...]).astype(jnp.float32)
    Gb = (iota_sub == r22_ref[...]).astype(jnp.float32)
    P = (iota_sub == p2_ref[...]).astype(jnp.float32)
    yB1 = jnp.dot(y_ref[...], Ga, preferred_element_type=jnp.float32)
    yB2 = jnp.dot(y_ref[...], Gb, preferred_element_type=jnp.float32)
    rb = c * yB1 * yB2 * den
    rb_ref[...] = rb
    w = jnp.sum((P - Ga - Gb) * cnt_ref[...], axis=0, keepdims=True)
    net_ref[...] += jnp.sum(rb * w, axis=1, keepdims=True)


def _scale(net1, net2, ys, ym):
    nl = _LF * (ys + ym)
    decay = jnp.minimum(_NAL / (nl + _EPS), 1.0)
    sc = decay * _sigmoid(net1 + net2)
    return jnp.log(sc)


def _s1st_kernel(ra_ref, p1_ref, r11_ref, k_ref, n1_ref, n2_ref,
                 ys_ref, ym_ref, dy_ref):
    i = pl.program_id(0)

    @pl.when(i == 0)
    def _init():
        dy_ref[...] = jnp.zeros_like(dy_ref)

    lsc = _scale(n1_ref[...], n2_ref[...], ys_ref[...], ym_ref[...])
    S1 = jnp.exp(k_ref[...] * lsc)
    rs = ra_ref[...] * S1
    iota_lane = jax.lax.broadcasted_iota(jnp.int32, (_C1, _S), 1)
    M = ((iota_lane == p1_ref[...]).astype(jnp.float32)
         - (iota_lane == r11_ref[...]).astype(jnp.float32))
    dy_ref[...] += jnp.dot(rs, M, preferred_element_type=jnp.float32)


def _s2nd_kernel(rb_ref, p2_ref, r12_ref, r22_ref, k_ref, n1_ref, n2_ref,
                 ys_ref, ym_ref, dy1_ref, dy_ref):
    i = pl.program_id(0)

    @pl.when(i == 0)
    def _init():
        dy_ref[...] = dy1_ref[...]

    lsc = _scale(n1_ref[...], n2_ref[...], ys_ref[...], ym_ref[...])
    S2 = jnp.exp(k_ref[...] * lsc)
    rs = rb_ref[...] * S2
    iota_lane = jax.lax.broadcasted_iota(jnp.int32, (_C2, _S), 1)
    M = ((iota_lane == p2_ref[...]).astype(jnp.float32)
         - (iota_lane == r12_ref[...]).astype(jnp.float32)
         - (iota_lane == r22_ref[...]).astype(jnp.float32))
    dy_ref[...] += jnp.dot(rs, M, preferred_element_type=jnp.float32)


def _row(x, n):
    return x.astype(jnp.int32).reshape(1, n)


def _col(x, n):
    return x.astype(jnp.int32).reshape(n, 1)


def kernel(t_in, y_in, alpha_1st, beta_1st, gamma_1st, alpha_2nd, beta_2nd,
           gamma_2nd, r1_1st, p_1st, r1_2nd, r2_2nd, p_2nd,
           inds_surf, inds_mant, inds_smt):
    f32 = jnp.float32
    t_col = t_in.astype(f32).reshape(_B, 1)
    y = y_in.astype(f32)
    a1 = alpha_1st.astype(f32).reshape(1, _R1)
    b1 = beta_1st.astype(f32).reshape(1, _R1)
    g1 = gamma_1st.astype(f32).reshape(1, _R1)
    a2 = alpha_2nd.astype(f32).reshape(1, _R2)
    b2 = beta_2nd.astype(f32).reshape(1, _R2)
    g2 = gamma_2nd.astype(f32).reshape(1, _R2)

    const = lambda *bs: pl.BlockSpec(bs, lambda i: (0,) * len(bs))
    rowblk = lambda c: pl.BlockSpec((1, c), lambda i: (0, i))
    colblk = lambda c: pl.BlockSpec((c, 1), lambda i: (i, 0))

    n1 = _R1 // _C1
    ra, net1, ysurf, ymant, cnt, kmat, dyp = pl.pallas_call(
        _p1st_kernel,
        grid=(n1,),
        in_specs=[
            const(_B, 1), const(_B, _S),
            rowblk(_C1), rowblk(_C1), rowblk(_C1),
            rowblk(_C1), rowblk(_C1),
            const(1, _NS), const(1, _NM),
            const(1, _NSMT), const(_NSMT, 1),
        ],
        out_specs=[
            pl.BlockSpec((_B, _C1), lambda i: (0, i)),
            const(_B, 1), const(_B, 1), const(_B, 1),
            const(_S, 1), const(_HI, 128), const(_B, _S),
        ],
        out_shape=[
            jax.ShapeDtypeStruct((_B, _R1), f32),
            jax.ShapeDtypeStruct((_B, 1), f32),
            jax.ShapeDtypeStruct((_B, 1), f32),
            jax.ShapeDtypeStruct((_B, 1), f32),
            jax.ShapeDtypeStruct((_S, 1), f32),
            jax.ShapeDtypeStruct((_HI, 128), f32),
            jax.ShapeDtypeStruct((_B, _S), f32),
        ],
        interpret=_INTERPRET,
    )(t_col, y, a1, b1, g1, _row(r1_1st, _R1), _row(p_1st, _R1),
      _row(inds_surf, _NS), _row(inds_mant, _NM),
      _row(inds_smt, _NSMT), _col(inds_smt, _NSMT))

    n2 = _R2 // _C2
    rb, net2 = pl.pallas_call(
        _p2nd_kernel,
        grid=(n2,),
        in_specs=[
            const(_B, 1), const(_B, _S),
            rowblk(_C2), rowblk(_C2), rowblk(_C2),
            rowblk(_C2), rowblk(_C2), rowblk(_C2),
            const(_S, 1),
        ],
        out_specs=[
            pl.BlockSpec((_B, _C2), lambda i: (0, i)),
            const(_B, 1),
        ],
        out_shape=[
            jax.ShapeDtypeStruct((_B, _R2), f32),
            jax.ShapeDtypeStruct((_B, 1), f32),
        ],
        interpret=_INTERPRET,
    )(t_col, y, a2, b2, g2, _row(r1_2nd, _R2), _row(r2_2nd, _R2),
      _row(p_2nd, _R2), cnt)

    k_row = kmat.reshape(1, _R1 + _R2)
    k1 = k_row[:, :_R1]
    k2 = k_row[:, _R1:]

    dy1 = pl.pallas_call(
        _s1st_kernel,
        grid=(n1,),
        in_specs=[
            pl.BlockSpec((_B, _C1), lambda i: (0, i)),
            colblk(_C1), colblk(_C1), rowblk(_C1),
            const(_B, 1), const(_B, 1), const(_B, 1), const(_B, 1),
        ],
        out_specs=const(_B, _S),
        out_shape=jax.ShapeDtypeStruct((_B, _S), f32),
        interpret=_INTERPRET,
    )(ra, _col(p_1st, _R1), _col(r1_1st, _R1), k1, net1, net2, ysurf, ymant)

    dy = pl.pallas_call(
        _s2nd_kernel,
        grid=(n2,),
        in_specs=[
            pl.BlockSpec((_B, _C2), lambda i: (0, i)),
            colblk(_C2), colblk(_C2), colblk(_C2), rowblk(_C2),
            const(_B, 1), const(_B, 1), const(_B, 1), const(_B, 1),
            const(_B, _S),
        ],
        out_specs=const(_B, _S),
        out_shape=jax.ShapeDtypeStruct((_B, _S), f32),
        interpret=_INTERPRET,
    )(rb, _col(p_2nd, _R2), _col(r1_2nd, _R2), _col(r2_2nd, _R2), k2,
      net1, net2, ysurf, ymant, dy1)

    return dy


# speedup vs baseline: 1.1401x; 1.1390x over previous
import functools

import jax
import jax.numpy as jnp
from jax import lax
from jax.experimental import pallas as pl
from jax.experimental.pallas import tpu as pltpu
from jax.experimental.pallas import tpu_sc as plsc

_B = 512
_S = 1024
_R1 = 8192
_R2 = 24576
_R = _R1 + _R2
_NS = 256
_NM = 256
_NSMT = 4096
_LF = 1e-6
_NAL = 2.0
_EPS = 1e-30

_C1 = 1024
_CP = 2048
_HI = _R // 128
_NW = 32
_EPW = _NSMT // _NW

_INTERPRET = False


def _sigmoid(x):
    return 1.0 / (1.0 + jnp.exp(-x))


def _med(t_col):
    Tg = 10.0 + 290.0 * _sigmoid(1e-3 * t_col)
    return jnp.log(Tg / 300.0), 1.0 / Tg


def _make_sc_gather():
    mesh = plsc.VectorSubcoreMesh(core_axis_name="c", subcore_axis_name="s")

    @functools.partial(
        pl.kernel,
        mesh=mesh,
        out_type=[
            jax.ShapeDtypeStruct((_NSMT,), jnp.float32),
            jax.ShapeDtypeStruct((_NSMT,), jnp.float32),
            jax.ShapeDtypeStruct((_NSMT,), jnp.float32),
            jax.ShapeDtypeStruct((_NSMT,), jnp.float32),
            jax.ShapeDtypeStruct((_NSMT,), jnp.int32),
            jax.ShapeDtypeStruct((_NSMT,), jnp.int32),
            jax.ShapeDtypeStruct((_NSMT,), jnp.int32),
        ],
        scratch_types=[
            pltpu.VMEM((_EPW,), jnp.int32),
            pltpu.VMEM((_EPW,), jnp.float32),
            pltpu.VMEM((_EPW,), jnp.int32),
            pltpu.SemaphoreType.DMA,
        ],
    )
    def _sc7(a_hbm, b_hbm, g_hbm, k_hbm, r1_hbm, r2_hbm, p_hbm, smt_hbm,
             ao_hbm, bo_hbm, go_hbm, ko_hbm, r1o_hbm, r2o_hbm, po_hbm,
             idx_v, fbuf, ibuf, sem):
        wid = lax.axis_index("s") * 2 + lax.axis_index("c")
        off = wid * _EPW
        pltpu.sync_copy(smt_hbm.at[pl.ds(off, _EPW)], idx_v)
        for src, dst, buf in ((a_hbm, ao_hbm, fbuf), (b_hbm, bo_hbm, fbuf),
                              (g_hbm, go_hbm, fbuf), (k_hbm, ko_hbm, fbuf),
                              (r1_hbm, r1o_hbm, ibuf), (r2_hbm, r2o_hbm, ibuf),
                              (p_hbm, po_hbm, ibuf)):
            pltpu.async_copy(src.at[idx_v], buf, sem).wait()
            pltpu.sync_copy(buf, dst.at[pl.ds(off, _EPW)])

    return _sc7


def _gather_params(acat, bcat, gcat, kflat, r1cat, r2cat, pcat, smt):
    return _make_sc_gather()(acat, bcat, gcat, kflat, r1cat, r2cat, pcat, smt)


def _p1st_kernel(t_ref, y_ref, a_ref, b_ref, g_ref, r11_ref, p1_ref,
                 surf_ref, mant_ref, smtr_ref, smtc_ref,
                 net_ref, ys_ref, ym_ref, cnt_ref, kmat_ref, dyp_ref):
    i = pl.program_id(0)

    @pl.when(i == 0)
    def _init():
        iota_s = jax.lax.broadcasted_iota(jnp.int32, (_S, _NS), 0)
        cnt = jnp.sum((iota_s == surf_ref[...]).astype(jnp.float32),
                      axis=1, keepdims=True)
        cntm = jnp.sum((iota_s == mant_ref[...]).astype(jnp.float32),
                       axis=1, keepdims=True)
        cnt_ref[...] = cnt
        ys_ref[...] = jnp.dot(y_ref[...], cnt,
                              preferred_element_type=jnp.float32)
        ym_ref[...] = jnp.dot(y_ref[...], cntm,
                              preferred_element_type=jnp.float32)
        hi_row = smtr_ref[...] // 128
        lo_col = smtc_ref[...] % 128
        mh = (jax.lax.broadcasted_iota(jnp.int32, (_HI, _NSMT), 0)
              == hi_row).astype(jnp.float32)
        ml = (jax.lax.broadcasted_iota(jnp.int32, (_NSMT, 128), 1)
              == lo_col).astype(jnp.float32)
        kmat_ref[...] = jnp.dot(mh, ml, preferred_element_type=jnp.float32)
        net_ref[...] = jnp.zeros_like(net_ref)
        dyp_ref[...] = jnp.zeros_like(dyp_ref)

    L, invT = _med(t_ref[...])
    c = a_ref[...] * jnp.exp(b_ref[...] * L - g_ref[...] * invT)
    iota_sub = jax.lax.broadcasted_iota(jnp.int32, (_S, _CP), 0)
    G = (iota_sub == r11_ref[...]).astype(jnp.float32)
    P = (iota_sub == p1_ref[...]).astype(jnp.float32)
    yA = jnp.dot(y_ref[...], G, preferred_element_type=jnp.float32)
    ra = c * yA
    PG = P - G
    w = jnp.sum(PG * cnt_ref[...], axis=0, keepdims=True)
    net_ref[...] += jnp.sum(ra * w, axis=1, keepdims=True)
    dyp_ref[...] += jax.lax.dot_general(
        ra, PG, (((1,), (1,)), ((), ())),
        preferred_element_type=jnp.float32)


def _p2nd_kernel(t_ref, y_ref, a_ref, b_ref, g_ref, r12_ref, r22_ref, p2_ref,
                 cnt_ref, dyp1_ref, net_ref, dyp_ref):
    i = pl.program_id(0)

    @pl.when(i == 0)
    def _init():
        net_ref[...] = jnp.zeros_like(net_ref)
        dyp_ref[...] = dyp1_ref[...]

    t = t_ref[...]
    L, invT = _med(t)
    den = jnp.exp(4.0 + 2.0 * jnp.tanh(5e-4 * t))
    c = a_ref[...] * jnp.exp(b_ref[...] * L - g_ref[...] * invT)
    iota_sub = jax.lax.broadcasted_iota(jnp.int32, (_S, _CP), 0)
    Ga = (iota_sub == r12_ref[...]).astype(jnp.float32)
    Gb = (iota_sub == r22_ref[...]).astype(jnp.float32)
    P = (iota_sub == p2_ref[...]).astype(jnp.float32)
    yB1 = jnp.dot(y_ref[...], Ga, preferred_element_type=jnp.float32)
    yB2 = jnp.dot(y_ref[...], Gb, preferred_element_type=jnp.float32)
    rb = c * yB1 * yB2 * den
    PG = P - Ga - Gb
    w = jnp.sum(PG * cnt_ref[...], axis=0, keepdims=True)
    net_ref[...] += jnp.sum(rb * w, axis=1, keepdims=True)
    dyp_ref[...] += jax.lax.dot_general(
        rb, PG, (((1,), (1,)), ((), ())),
        preferred_element_type=jnp.float32)


def _corr_kernel(t_ref, y_ref, as_ref, bs_ref, gs_ref, ks_ref,
                 r1r_ref, r2r_ref, smtr_ref,
                 pc_ref, r1c_ref, r2c_ref, smtc_ref,
                 n1_ref, n2_ref, ys_ref, ym_ref, dyp_ref, dy_ref):
    i = pl.program_id(0)

    @pl.when(i == 0)
    def _init():
        dy_ref[...] = dyp_ref[...]

    t = t_ref[...]
    nl = _LF * (ys_ref[...] + ym_ref[...])
    decay = jnp.minimum(_NAL / (nl + _EPS), 1.0)
    sc = decay * _sigmoid(n1_ref[...] + n2_ref[...])
    lsc = jnp.log(sc)
    L, invT = _med(t)
    ld = 4.0 + 2.0 * jnp.tanh(5e-4 * t)
    is2r = (smtr_ref[...] >= _R1).astype(jnp.float32)
    c = as_ref[...] * jnp.exp(bs_ref[...] * L - gs_ref[...] * invT
                              + is2r * ld)
    iota_sub = jax.lax.broadcasted_iota(jnp.int32, (_S, _C1), 0)
    G1 = (iota_sub == r1r_ref[...]).astype(jnp.float32)
    G2 = (iota_sub == r2r_ref[...]).astype(jnp.float32)
    yA = jnp.dot(y_ref[...], G1, preferred_element_type=jnp.float32)
    yB = jnp.dot(y_ref[...], G2, preferred_element_type=jnp.float32)
    yB = is2r * yB + (1.0 - is2r)
    k = ks_ref[...]
    scale = (jnp.exp(k * lsc) - 1.0) / k
    contrib = c * yA * yB * scale
    iota_lane = jax.lax.broadcasted_iota(jnp.int32, (_C1, _S), 1)
    is2c = (smtc_ref[...] >= _R1).astype(jnp.float32)
    M = ((iota_lane == pc_ref[...]).astype(jnp.float32)
         - (iota_lane == r1c_ref[...]).astype(jnp.float32)
         - is2c * (iota_lane == r2c_ref[...]).astype(jnp.float32))
    dy_ref[...] += jnp.dot(contrib, M, preferred_element_type=jnp.float32)


def _row(x, n):
    return x.astype(jnp.int32).reshape(1, n)


def _col(x, n):
    return x.astype(jnp.int32).reshape(n, 1)


def kernel(t_in, y_in, alpha_1st, beta_1st, gamma_1st, alpha_2nd, beta_2nd,
           gamma_2nd, r1_1st, p_1st, r1_2nd, r2_2nd, p_2nd,
           inds_surf, inds_mant, inds_smt):
    f32 = jnp.float32
    i32 = jnp.int32
    t_col = t_in.astype(f32).reshape(_B, 1)
    y = y_in.astype(f32)
    a1 = alpha_1st.astype(f32).reshape(1, _R1)
    b1 = beta_1st.astype(f32).reshape(1, _R1)
    g1 = gamma_1st.astype(f32).reshape(1, _R1)
    a2 = alpha_2nd.astype(f32).reshape(1, _R2)
    b2 = beta_2nd.astype(f32).reshape(1, _R2)
    g2 = gamma_2nd.astype(f32).reshape(1, _R2)

    const = lambda *bs: pl.BlockSpec(bs, lambda i: (0,) * len(bs))
    rowblk = lambda c: pl.BlockSpec((1, c), lambda i: (0, i))
    colblk = lambda c: pl.BlockSpec((c, 1), lambda i: (i, 0))

    n1 = _R1 // _CP
    net1, ysurf, ymant, cnt, kmat, dyp1 = pl.pallas_call(
        _p1st_kernel,
        grid=(n1,),
        in_specs=[
            const(_B, 1), const(_B, _S),
            rowblk(_CP), rowblk(_CP), rowblk(_CP),
            rowblk(_CP), rowblk(_CP),
            const(1, _NS), const(1, _NM),
            const(1, _NSMT), const(_NSMT, 1),
        ],
        out_specs=[
            const(_B, 1), const(_B, 1), const(_B, 1),
            const(_S, 1), const(_HI, 128), const(_B, _S),
        ],
        out_shape=[
            jax.ShapeDtypeStruct((_B, 1), f32),
            jax.ShapeDtypeStruct((_B, 1), f32),
            jax.ShapeDtypeStruct((_B, 1), f32),
            jax.ShapeDtypeStruct((_S, 1), f32),
            jax.ShapeDtypeStruct((_HI, 128), f32),
            jax.ShapeDtypeStruct((_B, _S), f32),
        ],
        interpret=_INTERPRET,
    )(t_col, y, a1, b1, g1, _row(r1_1st, _R1), _row(p_1st, _R1),
      _row(inds_surf, _NS), _row(inds_mant, _NM),
      _row(inds_smt, _NSMT), _col(inds_smt, _NSMT))

    n2 = _R2 // _CP
    net2, dyp = pl.pallas_call(
        _p2nd_kernel,
        grid=(n2,),
        in_specs=[
            const(_B, 1), const(_B, _S),
            rowblk(_CP), rowblk(_CP), rowblk(_CP),
            rowblk(_CP), rowblk(_CP), rowblk(_CP),
            const(_S, 1), const(_B, _S),
        ],
        out_specs=[
            const(_B, 1), const(_B, _S),
        ],
        out_shape=[
            jax.ShapeDtypeStruct((_B, 1), f32),
            jax.ShapeDtypeStruct((_B, _S), f32),
        ],
        interpret=_INTERPRET,
    )(t_col, y, a2, b2, g2, _row(r1_2nd, _R2), _row(r2_2nd, _R2),
      _row(p_2nd, _R2), cnt, dyp1)

    acat = jnp.concatenate([alpha_1st, alpha_2nd]).astype(f32)
    bcat = jnp.concatenate([beta_1st, beta_2nd]).astype(f32)
    gcat = jnp.concatenate([gamma_1st, gamma_2nd]).astype(f32)
    r1cat = jnp.concatenate([r1_1st, r1_2nd]).astype(i32)
    r2cat = jnp.concatenate([r1_1st, r2_2nd]).astype(i32)
    pcat = jnp.concatenate([p_1st, p_2nd]).astype(i32)
    kflat = kmat.reshape(_R)
    smt = inds_smt.astype(i32)

    a_s, b_s, g_s, k_s, r1_s, r2_s, p_s = _gather_params(
        acat, bcat, gcat, kflat, r1cat, r2cat, pcat, smt)

    nc = _NSMT // _C1
    dy = pl.pallas_call(
        _corr_kernel,
        grid=(nc,),
        in_specs=[
            const(_B, 1), const(_B, _S),
            rowblk(_C1), rowblk(_C1), rowblk(_C1), rowblk(_C1),
            rowblk(_C1), rowblk(_C1), rowblk(_C1),
            colblk(_C1), colblk(_C1), colblk(_C1), colblk(_C1),
            const(_B, 1), const(_B, 1), const(_B, 1), const(_B, 1),
            const(_B, _S),
        ],
        out_specs=const(_B, _S),
        out_shape=jax.ShapeDtypeStruct((_B, _S), f32),
        interpret=_INTERPRET,
    )(t_col, y,
      a_s.reshape(1, _NSMT), b_s.reshape(1, _NSMT), g_s.reshape(1, _NSMT),
      k_s.reshape(1, _NSMT),
      _row(r1_s, _NSMT), _row(r2_s, _NSMT), _row(smt, _NSMT),
      _col(p_s, _NSMT), _col(r1_s, _NSMT), _col(r2_s, _NSMT),
      _col(smt, _NSMT),
      net1, net2, ysurf, ymant, dyp)

    return dy


# speedup vs baseline: 1.2102x; 1.0614x over previous
import functools

import jax
import jax.numpy as jnp
from jax import lax
from jax.experimental import pallas as pl
from jax.experimental.pallas import tpu as pltpu
from jax.experimental.pallas import tpu_sc as plsc

_B = 512
_S = 1024
_R1 = 8192
_R2 = 24576
_R = _R1 + _R2
_NS = 256
_NM = 256
_NSMT = 4096
_LF = 1e-6
_NAL = 2.0
_EPS = 1e-30

_C1 = 1024
_CP = 2048
_CC = 2048
_HI = _R // 128
_NW = 32
_EPW = _NSMT // _NW

_INTERPRET = False


def _sigmoid(x):
    return 1.0 / (1.0 + jnp.exp(-x))


def _med(t_col):
    Tg = 10.0 + 290.0 * _sigmoid(1e-3 * t_col)
    return jnp.log(Tg / 300.0), 1.0 / Tg


def _make_sc_gather():
    mesh = plsc.VectorSubcoreMesh(core_axis_name="c", subcore_axis_name="s")

    @functools.partial(
        pl.kernel,
        mesh=mesh,
        out_type=[
            jax.ShapeDtypeStruct((_NSMT,), jnp.float32),
            jax.ShapeDtypeStruct((_NSMT,), jnp.float32),
            jax.ShapeDtypeStruct((_NSMT,), jnp.float32),
            jax.ShapeDtypeStruct((_NSMT,), jnp.float32),
            jax.ShapeDtypeStruct((_NSMT,), jnp.int32),
            jax.ShapeDtypeStruct((_NSMT,), jnp.int32),
            jax.ShapeDtypeStruct((_NSMT,), jnp.int32),
        ],
        scratch_types=[
            pltpu.VMEM((_EPW,), jnp.int32),
            pltpu.VMEM((_EPW,), jnp.float32),
            pltpu.VMEM((_EPW,), jnp.int32),
            pltpu.SemaphoreType.DMA,
        ],
    )
    def _sc7(a_hbm, b_hbm, g_hbm, k_hbm, r1_hbm, r2_hbm, p_hbm, smt_hbm,
             ao_hbm, bo_hbm, go_hbm, ko_hbm, r1o_hbm, r2o_hbm, po_hbm,
             idx_v, fbuf, ibuf, sem):
        wid = lax.axis_index("s") * 2 + lax.axis_index("c")
        off = wid * _EPW
        pltpu.sync_copy(smt_hbm.at[pl.ds(off, _EPW)], idx_v)
        for src, dst, buf in ((a_hbm, ao_hbm, fbuf), (b_hbm, bo_hbm, fbuf),
                              (g_hbm, go_hbm, fbuf), (k_hbm, ko_hbm, fbuf),
                              (r1_hbm, r1o_hbm, ibuf), (r2_hbm, r2o_hbm, ibuf),
                              (p_hbm, po_hbm, ibuf)):
            pltpu.async_copy(src.at[idx_v], buf, sem).wait()
            pltpu.sync_copy(buf, dst.at[pl.ds(off, _EPW)])

    return _sc7


def _gather_params(acat, bcat, gcat, kflat, r1cat, r2cat, pcat, smt):
    return _make_sc_gather()(acat, bcat, gcat, kflat, r1cat, r2cat, pcat, smt)


def _p1st_kernel(t_ref, y_ref, a_ref, b_ref, g_ref, r11_ref, p1_ref,
                 surf_ref, mant_ref, smtr_ref, smtc_ref,
                 net_ref, ys_ref, ym_ref, cnt_ref, kmat_ref, dyp_ref):
    i = pl.program_id(0)

    @pl.when(i == 0)
    def _init():
        iota_s = jax.lax.broadcasted_iota(jnp.int32, (_S, _NS), 0)
        cnt = jnp.sum((iota_s == surf_ref[...]).astype(jnp.float32),
                      axis=1, keepdims=True)
        cntm = jnp.sum((iota_s == mant_ref[...]).astype(jnp.float32),
                       axis=1, keepdims=True)
        cnt_ref[...] = cnt
        ys_ref[...] = jnp.dot(y_ref[...], cnt,
                              preferred_element_type=jnp.float32)
        ym_ref[...] = jnp.dot(y_ref[...], cntm,
                              preferred_element_type=jnp.float32)
        hi_row = smtr_ref[...] // 128
        lo_col = smtc_ref[...] % 128
        mh = (jax.lax.broadcasted_iota(jnp.int32, (_HI, _NSMT), 0)
              == hi_row).astype(jnp.float32)
        ml = (jax.lax.broadcasted_iota(jnp.int32, (_NSMT, 128), 1)
              == lo_col).astype(jnp.float32)
        kmat_ref[...] = jnp.dot(mh, ml, preferred_element_type=jnp.float32)
        net_ref[...] = jnp.zeros_like(net_ref)
        dyp_ref[...] = jnp.zeros_like(dyp_ref)

    L, invT = _med(t_ref[...])
    c = a_ref[...] * jnp.exp(b_ref[...] * L - g_ref[...] * invT)
    iota_sub = jax.lax.broadcasted_iota(jnp.int32, (_S, _CP), 0)
    G = (iota_sub == r11_ref[...]).astype(jnp.float32)
    P = (iota_sub == p1_ref[...]).astype(jnp.float32)
    yA = jnp.dot(y_ref[...], G, preferred_element_type=jnp.float32)
    ra = c * yA
    PG = P - G
    w = jnp.sum(PG * cnt_ref[...], axis=0, keepdims=True)
    net_ref[...] += jnp.sum(ra * w, axis=1, keepdims=True)
    dyp_ref[...] += jax.lax.dot_general(
        ra, PG, (((1,), (1,)), ((), ())),
        preferred_element_type=jnp.float32)


def _p2nd_kernel(t_ref, y_ref, a_ref, b_ref, g_ref, r12_ref, r22_ref, p2_ref,
                 cnt_ref, dyp1_ref, net_ref, dyp_ref):
    i = pl.program_id(0)

    @pl.when(i == 0)
    def _init():
        net_ref[...] = jnp.zeros_like(net_ref)
        dyp_ref[...] = dyp1_ref[...]

    t = t_ref[...]
    L, invT = _med(t)
    den = jnp.exp(4.0 + 2.0 * jnp.tanh(5e-4 * t))
    c = a_ref[...] * jnp.exp(b_ref[...] * L - g_ref[...] * invT)
    iota_sub = jax.lax.broadcasted_iota(jnp.int32, (_S, _CP), 0)
    Ga = (iota_sub == r12_ref[...]).astype(jnp.float32)
    Gb = (iota_sub == r22_ref[...]).astype(jnp.float32)
    P = (iota_sub == p2_ref[...]).astype(jnp.float32)
    yB1 = jnp.dot(y_ref[...], Ga, preferred_element_type=jnp.float32)
    yB2 = jnp.dot(y_ref[...], Gb, preferred_element_type=jnp.float32)
    rb = c * yB1 * yB2 * den
    PG = P - Ga - Gb
    w = jnp.sum(PG * cnt_ref[...], axis=0, keepdims=True)
    net_ref[...] += jnp.sum(rb * w, axis=1, keepdims=True)
    dyp_ref[...] += jax.lax.dot_general(
        rb, PG, (((1,), (1,)), ((), ())),
        preferred_element_type=jnp.float32)


def _corr_kernel(t_ref, y_ref, as_ref, bs_ref, gs_ref, ks_ref,
                 r1r_ref, r2r_ref, pr_ref, smtr_ref,
                 n1_ref, n2_ref, ys_ref, ym_ref, dyp_ref, dy_ref):
    i = pl.program_id(0)

    @pl.when(i == 0)
    def _init():
        dy_ref[...] = dyp_ref[...]

    t = t_ref[...]
    nl = _LF * (ys_ref[...] + ym_ref[...])
    decay = jnp.minimum(_NAL / (nl + _EPS), 1.0)
    sc = decay * _sigmoid(n1_ref[...] + n2_ref[...])
    lsc = jnp.log(sc)
    L, invT = _med(t)
    ld = 4.0 + 2.0 * jnp.tanh(5e-4 * t)
    is2r = (smtr_ref[...] >= _R1).astype(jnp.float32)
    c = as_ref[...] * jnp.exp(bs_ref[...] * L - gs_ref[...] * invT
                              + is2r * ld)
    iota_sub = jax.lax.broadcasted_iota(jnp.int32, (_S, _CC), 0)
    G1 = (iota_sub == r1r_ref[...]).astype(jnp.float32)
    G2 = (iota_sub == r2r_ref[...]).astype(jnp.float32)
    yA = jnp.dot(y_ref[...], G1, preferred_element_type=jnp.float32)
    yB = jnp.dot(y_ref[...], G2, preferred_element_type=jnp.float32)
    yB = is2r * yB + (1.0 - is2r)
    k = ks_ref[...]
    scale = (jnp.exp(k * lsc) - 1.0) / k
    contrib = c * yA * yB * scale
    Pt = (iota_sub == pr_ref[...]).astype(jnp.float32)
    Mt = Pt - G1 - is2r * G2
    dy_ref[...] += jax.lax.dot_general(
        contrib, Mt, (((1,), (1,)), ((), ())),
        preferred_element_type=jnp.float32)


def _row(x, n):
    return x.astype(jnp.int32).reshape(1, n)


def _col(x, n):
    return x.astype(jnp.int32).reshape(n, 1)


def kernel(t_in, y_in, alpha_1st, beta_1st, gamma_1st, alpha_2nd, beta_2nd,
           gamma_2nd, r1_1st, p_1st, r1_2nd, r2_2nd, p_2nd,
           inds_surf, inds_mant, inds_smt):
    f32 = jnp.float32
    i32 = jnp.int32
    t_col = t_in.astype(f32).reshape(_B, 1)
    y = y_in.astype(f32)
    a1 = alpha_1st.astype(f32).reshape(1, _R1)
    b1 = beta_1st.astype(f32).reshape(1, _R1)
    g1 = gamma_1st.astype(f32).reshape(1, _R1)
    a2 = alpha_2nd.astype(f32).reshape(1, _R2)
    b2 = beta_2nd.astype(f32).reshape(1, _R2)
    g2 = gamma_2nd.astype(f32).reshape(1, _R2)

    const = lambda *bs: pl.BlockSpec(bs, lambda i: (0,) * len(bs))
    rowblk = lambda c: pl.BlockSpec((1, c), lambda i: (0, i))
    colblk = lambda c: pl.BlockSpec((c, 1), lambda i: (i, 0))

    n1 = _R1 // _CP
    net1, ysurf, ymant, cnt, kmat, dyp1 = pl.pallas_call(
        _p1st_kernel,
        grid=(n1,),
        in_specs=[
            const(_B, 1), const(_B, _S),
            rowblk(_CP), rowblk(_CP), rowblk(_CP),
            rowblk(_CP), rowblk(_CP),
            const(1, _NS), const(1, _NM),
            const(1, _NSMT), const(_NSMT, 1),
        ],
        out_specs=[
            const(_B, 1), const(_B, 1), const(_B, 1),
            const(_S, 1), const(_HI, 128), const(_B, _S),
        ],
        out_shape=[
            jax.ShapeDtypeStruct((_B, 1), f32),
            jax.ShapeDtypeStruct((_B, 1), f32),
            jax.ShapeDtypeStruct((_B, 1), f32),
            jax.ShapeDtypeStruct((_S, 1), f32),
            jax.ShapeDtypeStruct((_HI, 128), f32),
            jax.ShapeDtypeStruct((_B, _S), f32),
        ],
        interpret=_INTERPRET,
    )(t_col, y, a1, b1, g1, _row(r1_1st, _R1), _row(p_1st, _R1),
      _row(inds_surf, _NS), _row(inds_mant, _NM),
      _row(inds_smt, _NSMT), _col(inds_smt, _NSMT))

    n2 = _R2 // _CP
    net2, dyp = pl.pallas_call(
        _p2nd_kernel,
        grid=(n2,),
        in_specs=[
            const(_B, 1), const(_B, _S),
            rowblk(_CP), rowblk(_CP), rowblk(_CP),
            rowblk(_CP), rowblk(_CP), rowblk(_CP),
            const(_S, 1), const(_B, _S),
        ],
        out_specs=[
            const(_B, 1), const(_B, _S),
        ],
        out_shape=[
            jax.ShapeDtypeStruct((_B, 1), f32),
            jax.ShapeDtypeStruct((_B, _S), f32),
        ],
        interpret=_INTERPRET,
    )(t_col, y, a2, b2, g2, _row(r1_2nd, _R2), _row(r2_2nd, _R2),
      _row(p_2nd, _R2), cnt, dyp1)

    acat = jnp.concatenate([alpha_1st, alpha_2nd]).astype(f32)
    bcat = jnp.concatenate([beta_1st, beta_2nd]).astype(f32)
    gcat = jnp.concatenate([gamma_1st, gamma_2nd]).astype(f32)
    r1cat = jnp.concatenate([r1_1st, r1_2nd]).astype(i32)
    r2cat = jnp.concatenate([r1_1st, r2_2nd]).astype(i32)
    pcat = jnp.concatenate([p_1st, p_2nd]).astype(i32)
    kflat = kmat.reshape(_R)
    smt = inds_smt.astype(i32)

    a_s, b_s, g_s, k_s, r1_s, r2_s, p_s = _gather_params(
        acat, bcat, gcat, kflat, r1cat, r2cat, pcat, smt)

    nc = _NSMT // _CC
    dy = pl.pallas_call(
        _corr_kernel,
        grid=(nc,),
        in_specs=[
            const(_B, 1), const(_B, _S),
            rowblk(_CC), rowblk(_CC), rowblk(_CC), rowblk(_CC),
            rowblk(_CC), rowblk(_CC), rowblk(_CC), rowblk(_CC),
            const(_B, 1), const(_B, 1), const(_B, 1), const(_B, 1),
            const(_B, _S),
        ],
        out_specs=const(_B, _S),
        out_shape=jax.ShapeDtypeStruct((_B, _S), f32),
        interpret=_INTERPRET,
    )(t_col, y,
      a_s.reshape(1, _NSMT), b_s.reshape(1, _NSMT), g_s.reshape(1, _NSMT),
      k_s.reshape(1, _NSMT),
      _row(r1_s, _NSMT), _row(r2_s, _NSMT), _row(p_s, _NSMT),
      _row(smt, _NSMT),
      net1, net2, ysurf, ymant, dyp)

    return dy


# speedup vs baseline: 1.3382x; 1.1058x over previous
import functools

import jax
import jax.numpy as jnp
from jax import lax
from jax.experimental import pallas as pl
from jax.experimental.pallas import tpu as pltpu
from jax.experimental.pallas import tpu_sc as plsc

_B = 512
_S = 1024
_R1 = 8192
_R2 = 24576
_R = _R1 + _R2
_NS = 256
_NM = 256
_NSMT = 4096
_LF = 1e-6
_NAL = 2.0
_EPS = 1e-30

_C1 = 1024
_CP = 2048
_CC = 2048
_HI = _R // 128
_NW = 32
_EPW = _NSMT // _NW

_INTERPRET = False


def _sigmoid(x):
    return 1.0 / (1.0 + jnp.exp(-x))


def _med(t_col):
    Tg = 10.0 + 290.0 * _sigmoid(1e-3 * t_col)
    return jnp.log(Tg / 300.0), 1.0 / Tg


def _make_sc_gather():
    mesh = plsc.VectorSubcoreMesh(core_axis_name="c", subcore_axis_name="s")

    @functools.partial(
        pl.kernel,
        mesh=mesh,
        out_type=[
            jax.ShapeDtypeStruct((_NSMT,), jnp.float32),
            jax.ShapeDtypeStruct((_NSMT,), jnp.float32),
            jax.ShapeDtypeStruct((_NSMT,), jnp.float32),
            jax.ShapeDtypeStruct((_NSMT,), jnp.float32),
            jax.ShapeDtypeStruct((_NSMT,), jnp.int32),
            jax.ShapeDtypeStruct((_NSMT,), jnp.int32),
            jax.ShapeDtypeStruct((_NSMT,), jnp.int32),
        ],
        scratch_types=[
            pltpu.VMEM((_EPW,), jnp.int32),
            pltpu.VMEM((_EPW,), jnp.float32),
            pltpu.VMEM((_EPW,), jnp.int32),
            pltpu.SemaphoreType.DMA,
        ],
    )
    def _sc7(a_hbm, b_hbm, g_hbm, k_hbm, r1_hbm, r2_hbm, p_hbm, smt_hbm,
             ao_hbm, bo_hbm, go_hbm, ko_hbm, r1o_hbm, r2o_hbm, po_hbm,
             idx_v, fbuf, ibuf, sem):
        wid = lax.axis_index("s") * 2 + lax.axis_index("c")
        off = wid * _EPW
        pltpu.sync_copy(smt_hbm.at[pl.ds(off, _EPW)], idx_v)
        for src, dst, buf in ((a_hbm, ao_hbm, fbuf), (b_hbm, bo_hbm, fbuf),
                              (g_hbm, go_hbm, fbuf), (k_hbm, ko_hbm, fbuf),
                              (r1_hbm, r1o_hbm, ibuf), (r2_hbm, r2o_hbm, ibuf),
                              (p_hbm, po_hbm, ibuf)):
            pltpu.async_copy(src.at[idx_v], buf, sem).wait()
            pltpu.sync_copy(buf, dst.at[pl.ds(off, _EPW)])

    return _sc7


def _gather_params(acat, bcat, gcat, kflat, r1cat, r2cat, pcat, smt):
    return _make_sc_gather()(acat, bcat, gcat, kflat, r1cat, r2cat, pcat, smt)


def _p1st_kernel(t_ref, y_ref, a_ref, b_ref, g_ref, r11_ref, p1_ref,
                 surf_ref, mant_ref, smtr_ref, smtc_ref,
                 ys_ref, ym_ref, cnt_ref, kmat_ref, dyp_ref):
    i = pl.program_id(0)

    @pl.when(i == 0)
    def _init():
        iota_s = jax.lax.broadcasted_iota(jnp.int32, (_S, _NS), 0)
        cnt = jnp.sum((iota_s == surf_ref[...]).astype(jnp.float32),
                      axis=1, keepdims=True)
        cntm = jnp.sum((iota_s == mant_ref[...]).astype(jnp.float32),
                       axis=1, keepdims=True)
        cnt_ref[...] = cnt
        ys_ref[...] = jnp.dot(y_ref[...], cnt,
                              preferred_element_type=jnp.float32)
        ym_ref[...] = jnp.dot(y_ref[...], cntm,
                              preferred_element_type=jnp.float32)
        hi_row = smtr_ref[...] // 128
        lo_col = smtc_ref[...] % 128
        mh = (jax.lax.broadcasted_iota(jnp.int32, (_HI, _NSMT), 0)
              == hi_row).astype(jnp.float32)
        ml = (jax.lax.broadcasted_iota(jnp.int32, (_NSMT, 128), 1)
              == lo_col).astype(jnp.float32)
        kmat_ref[...] = jnp.dot(mh, ml, preferred_element_type=jnp.float32)
        dyp_ref[...] = jnp.zeros_like(dyp_ref)

    L, invT = _med(t_ref[...])
    c = a_ref[...] * jnp.exp(b_ref[...] * L - g_ref[...] * invT)
    iota_sub = jax.lax.broadcasted_iota(jnp.int32, (_S, _CP), 0)
    G = (iota_sub == r11_ref[...]).astype(jnp.float32)
    P = (iota_sub == p1_ref[...]).astype(jnp.float32)
    yA = jnp.dot(y_ref[...], G, preferred_element_type=jnp.float32)
    ra = c * yA
    PG = P - G
    dyp_ref[...] += jax.lax.dot_general(
        ra, PG, (((1,), (1,)), ((), ())),
        preferred_element_type=jnp.float32)


def _p2nd_kernel(t_ref, y_ref, a_ref, b_ref, g_ref, r12_ref, r22_ref, p2_ref,
                 dyp1_ref, dyp_ref):
    i = pl.program_id(0)

    @pl.when(i == 0)
    def _init():
        dyp_ref[...] = dyp1_ref[...]

    t = t_ref[...]
    L, invT = _med(t)
    den = jnp.exp(4.0 + 2.0 * jnp.tanh(5e-4 * t))
    c = a_ref[...] * jnp.exp(b_ref[...] * L - g_ref[...] * invT)
    iota_sub = jax.lax.broadcasted_iota(jnp.int32, (_S, _CP), 0)
    Ga = (iota_sub == r12_ref[...]).astype(jnp.float32)
    Gb = (iota_sub == r22_ref[...]).astype(jnp.float32)
    P = (iota_sub == p2_ref[...]).astype(jnp.float32)
    yB1 = jnp.dot(y_ref[...], Ga, preferred_element_type=jnp.float32)
    yB2 = jnp.dot(y_ref[...], Gb, preferred_element_type=jnp.float32)
    rb = c * yB1 * yB2 * den
    PG = P - Ga - Gb
    dyp_ref[...] += jax.lax.dot_general(
        rb, PG, (((1,), (1,)), ((), ())),
        preferred_element_type=jnp.float32)


def _corr_kernel(t_ref, y_ref, as_ref, bs_ref, gs_ref, ks_ref,
                 r1r_ref, r2r_ref, pr_ref, smtr_ref,
                 cnt_ref, ys_ref, ym_ref, dyp_ref, dy_ref):
    i = pl.program_id(0)

    @pl.when(i == 0)
    def _init():
        dy_ref[...] = dyp_ref[...]

    t = t_ref[...]
    nl = _LF * (ys_ref[...] + ym_ref[...])
    decay = jnp.minimum(_NAL / (nl + _EPS), 1.0)
    net = jnp.dot(dyp_ref[...], cnt_ref[...],
                  preferred_element_type=jnp.float32)
    sc = decay * _sigmoid(net)
    lsc = jnp.log(sc)
    L, invT = _med(t)
    ld = 4.0 + 2.0 * jnp.tanh(5e-4 * t)
    is2r = (smtr_ref[...] >= _R1).astype(jnp.float32)
    c = as_ref[...] * jnp.exp(bs_ref[...] * L - gs_ref[...] * invT
                              + is2r * ld)
    iota_sub = jax.lax.broadcasted_iota(jnp.int32, (_S, _CC), 0)
    G1 = (iota_sub == r1r_ref[...]).astype(jnp.float32)
    G2 = (iota_sub == r2r_ref[...]).astype(jnp.float32)
    yA = jnp.dot(y_ref[...], G1, preferred_element_type=jnp.float32)
    yB = jnp.dot(y_ref[...], G2, preferred_element_type=jnp.float32)
    yB = is2r * yB + (1.0 - is2r)
    k = ks_ref[...]
    scale = (jnp.exp(k * lsc) - 1.0) / k
    contrib = c * yA * yB * scale
    Pt = (iota_sub == pr_ref[...]).astype(jnp.float32)
    Mt = Pt - G1 - is2r * G2
    dy_ref[...] += jax.lax.dot_general(
        contrib, Mt, (((1,), (1,)), ((), ())),
        preferred_element_type=jnp.float32)


def _row(x, n):
    return x.astype(jnp.int32).reshape(1, n)


def _col(x, n):
    return x.astype(jnp.int32).reshape(n, 1)


def kernel(t_in, y_in, alpha_1st, beta_1st, gamma_1st, alpha_2nd, beta_2nd,
           gamma_2nd, r1_1st, p_1st, r1_2nd, r2_2nd, p_2nd,
           inds_surf, inds_mant, inds_smt):
    f32 = jnp.float32
    i32 = jnp.int32
    t_col = t_in.astype(f32).reshape(_B, 1)
    y = y_in.astype(f32)
    a1 = alpha_1st.astype(f32).reshape(1, _R1)
    b1 = beta_1st.astype(f32).reshape(1, _R1)
    g1 = gamma_1st.astype(f32).reshape(1, _R1)
    a2 = alpha_2nd.astype(f32).reshape(1, _R2)
    b2 = beta_2nd.astype(f32).reshape(1, _R2)
    g2 = gamma_2nd.astype(f32).reshape(1, _R2)

    const = lambda *bs: pl.BlockSpec(bs, lambda i: (0,) * len(bs))
    rowblk = lambda c: pl.BlockSpec((1, c), lambda i: (0, i))
    colblk = lambda c: pl.BlockSpec((c, 1), lambda i: (i, 0))

    n1 = _R1 // _CP
    ysurf, ymant, cnt, kmat, dyp1 = pl.pallas_call(
        _p1st_kernel,
        grid=(n1,),
        in_specs=[
            const(_B, 1), const(_B, _S),
            rowblk(_CP), rowblk(_CP), rowblk(_CP),
            rowblk(_CP), rowblk(_CP),
            const(1, _NS), const(1, _NM),
            const(1, _NSMT), const(_NSMT, 1),
        ],
        out_specs=[
            const(_B, 1), const(_B, 1),
            const(_S, 1), const(_HI, 128), const(_B, _S),
        ],
        out_shape=[
            jax.ShapeDtypeStruct((_B, 1), f32),
            jax.ShapeDtypeStruct((_B, 1), f32),
            jax.ShapeDtypeStruct((_S, 1), f32),
            jax.ShapeDtypeStruct((_HI, 128), f32),
            jax.ShapeDtypeStruct((_B, _S), f32),
        ],
        interpret=_INTERPRET,
    )(t_col, y, a1, b1, g1, _row(r1_1st, _R1), _row(p_1st, _R1),
      _row(inds_surf, _NS), _row(inds_mant, _NM),
      _row(inds_smt, _NSMT), _col(inds_smt, _NSMT))

    n2 = _R2 // _CP
    dyp = pl.pallas_call(
        _p2nd_kernel,
        grid=(n2,),
        in_specs=[
            const(_B, 1), const(_B, _S),
            rowblk(_CP), rowblk(_CP), rowblk(_CP),
            rowblk(_CP), rowblk(_CP), rowblk(_CP),
            const(_B, _S),
        ],
        out_specs=const(_B, _S),
        out_shape=jax.ShapeDtypeStruct((_B, _S), f32),
        interpret=_INTERPRET,
    )(t_col, y, a2, b2, g2, _row(r1_2nd, _R2), _row(r2_2nd, _R2),
      _row(p_2nd, _R2), dyp1)

    acat = jnp.concatenate([alpha_1st, alpha_2nd]).astype(f32)
    bcat = jnp.concatenate([beta_1st, beta_2nd]).astype(f32)
    gcat = jnp.concatenate([gamma_1st, gamma_2nd]).astype(f32)
    r1cat = jnp.concatenate([r1_1st, r1_2nd]).astype(i32)
    r2cat = jnp.concatenate([r1_1st, r2_2nd]).astype(i32)
    pcat = jnp.concatenate([p_1st, p_2nd]).astype(i32)
    kflat = kmat.reshape(_R)
    smt = inds_smt.astype(i32)

    a_s, b_s, g_s, k_s, r1_s, r2_s, p_s = _gather_params(
        acat, bcat, gcat, kflat, r1cat, r2cat, pcat, smt)

    nc = _NSMT // _CC
    dy = pl.pallas_call(
        _corr_kernel,
        grid=(nc,),
        in_specs=[
            const(_B, 1), const(_B, _S),
            rowblk(_CC), rowblk(_CC), rowblk(_CC), rowblk(_CC),
            rowblk(_CC), rowblk(_CC), rowblk(_CC), rowblk(_CC),
            const(_S, 1), const(_B, 1), const(_B, 1),
            const(_B, _S),
        ],
        out_specs=const(_B, _S),
        out_shape=jax.ShapeDtypeStruct((_B, _S), f32),
        interpret=_INTERPRET,
    )(t_col, y,
      a_s.reshape(1, _NSMT), b_s.reshape(1, _NSMT), g_s.reshape(1, _NSMT),
      k_s.reshape(1, _NSMT),
      _row(r1_s, _NSMT), _row(r2_s, _NSMT), _row(p_s, _NSMT),
      _row(smt, _NSMT),
      cnt, ysurf, ymant, dyp)

    return dy


# speedup vs baseline: 1.3387x; 1.0003x over previous
import functools

import jax
import jax.numpy as jnp
from jax import lax
from jax.experimental import pallas as pl
from jax.experimental.pallas import tpu as pltpu
from jax.experimental.pallas import tpu_sc as plsc

_B = 512
_S = 1024
_R1 = 8192
_R2 = 24576
_R = _R1 + _R2
_NS = 256
_NM = 256
_NSMT = 4096
_LF = 1e-6
_NAL = 2.0
_EPS = 1e-30

_C1 = 1024
_CP = 2048
_CC = 2048
_HI = _R // 128
_NW = 32
_EPW = _NSMT // _NW


def _sigmoid(x):
    return 1.0 / (1.0 + jnp.exp(-x))


def _med(t_col):
    Tg = 10.0 + 290.0 * _sigmoid(1e-3 * t_col)
    return jnp.log(Tg / 300.0), 1.0 / Tg


def _make_sc_gather():
    mesh = plsc.VectorSubcoreMesh(core_axis_name="c", subcore_axis_name="s")

    @functools.partial(
        pl.kernel,
        mesh=mesh,
        out_type=[
            jax.ShapeDtypeStruct((_NSMT,), jnp.float32),
            jax.ShapeDtypeStruct((_NSMT,), jnp.float32),
            jax.ShapeDtypeStruct((_NSMT,), jnp.float32),
            jax.ShapeDtypeStruct((_NSMT,), jnp.float32),
            jax.ShapeDtypeStruct((_NSMT,), jnp.int32),
            jax.ShapeDtypeStruct((_NSMT,), jnp.int32),
            jax.ShapeDtypeStruct((_NSMT,), jnp.int32),
        ],
        scratch_types=[
            pltpu.VMEM((_EPW,), jnp.int32),
            pltpu.VMEM((_EPW,), jnp.float32),
            pltpu.VMEM((_EPW,), jnp.int32),
            pltpu.SemaphoreType.DMA,
        ],
    )
    def _sc7(a_hbm, b_hbm, g_hbm, k_hbm, r1_hbm, r2_hbm, p_hbm, smt_hbm,
             ao_hbm, bo_hbm, go_hbm, ko_hbm, r1o_hbm, r2o_hbm, po_hbm,
             idx_v, fbuf, ibuf, sem):
        wid = lax.axis_index("s") * 2 + lax.axis_index("c")
        off = wid * _EPW
        pltpu.sync_copy(smt_hbm.at[pl.ds(off, _EPW)], idx_v)
        for src, dst, buf in ((a_hbm, ao_hbm, fbuf), (b_hbm, bo_hbm, fbuf),
                              (g_hbm, go_hbm, fbuf), (k_hbm, ko_hbm, fbuf),
                              (r1_hbm, r1o_hbm, ibuf), (r2_hbm, r2o_hbm, ibuf),
                              (p_hbm, po_hbm, ibuf)):
            pltpu.async_copy(src.at[idx_v], buf, sem).wait()
            pltpu.sync_copy(buf, dst.at[pl.ds(off, _EPW)])

    return _sc7


def _gather_params(acat, bcat, gcat, kflat, r1cat, r2cat, pcat, smt):
    return _make_sc_gather()(acat, bcat, gcat, kflat, r1cat, r2cat, pcat, smt)


def _p1st_kernel(t_ref, y_ref, a_ref, b_ref, g_ref, r11_ref, p1_ref,
                 surf_ref, mant_ref, smtr_ref, smtc_ref,
                 ys_ref, ym_ref, cnt_ref, kmat_ref, dyp_ref):
    i = pl.program_id(0)

    @pl.when(i == 0)
    def _init():
        iota_s = jax.lax.broadcasted_iota(jnp.int32, (_S, _NS), 0)
        cnt = jnp.sum((iota_s == surf_ref[...]).astype(jnp.float32),
                      axis=1, keepdims=True)
        cntm = jnp.sum((iota_s == mant_ref[...]).astype(jnp.float32),
                       axis=1, keepdims=True)
        cnt_ref[...] = cnt
        ys_ref[...] = jnp.dot(y_ref[...], cnt,
                              preferred_element_type=jnp.float32)
        ym_ref[...] = jnp.dot(y_ref[...], cntm,
                              preferred_element_type=jnp.float32)
        hi_row = smtr_ref[...] // 128
        lo_col = smtc_ref[...] % 128
        mh = (jax.lax.broadcasted_iota(jnp.int32, (_HI, _NSMT), 0)
              == hi_row).astype(jnp.float32)
        ml = (jax.lax.broadcasted_iota(jnp.int32, (_NSMT, 128), 1)
              == lo_col).astype(jnp.float32)
        kmat_ref[...] = jnp.dot(mh, ml, preferred_element_type=jnp.float32)
        dyp_ref[...] = jnp.zeros_like(dyp_ref)

    L, invT = _med(t_ref[...])
    c = a_ref[...] * jnp.exp(b_ref[...] * L - g_ref[...] * invT)
    iota_sub = jax.lax.broadcasted_iota(jnp.int32, (_S, _CP), 0)
    G = (iota_sub == r11_ref[...]).astype(jnp.float32)
    P = (iota_sub == p1_ref[...]).astype(jnp.float32)
    yA = jnp.dot(y_ref[...], G, preferred_element_type=jnp.float32)
    ra = c * yA
    PG = P - G
    dyp_ref[...] += jax.lax.dot_general(
        ra, PG, (((1,), (1,)), ((), ())),
        preferred_element_type=jnp.float32)


def _p2nd_kernel(t_ref, y_ref, a_ref, b_ref, g_ref, r12_ref, r22_ref, p2_ref,
                 dyp1_ref, dyp_ref):
    i = pl.program_id(0)

    @pl.when(i == 0)
    def _init():
        dyp_ref[...] = dyp1_ref[...]

    t = t_ref[...]
    L, invT = _med(t)
    den = jnp.exp(4.0 + 2.0 * jnp.tanh(5e-4 * t))
    c = a_ref[...] * jnp.exp(b_ref[...] * L - g_ref[...] * invT)
    iota_sub = jax.lax.broadcasted_iota(jnp.int32, (_S, _CP), 0)
    Ga = (iota_sub == r12_ref[...]).astype(jnp.float32)
    Gb = (iota_sub == r22_ref[...]).astype(jnp.float32)
    P = (iota_sub == p2_ref[...]).astype(jnp.float32)
    yB1 = jnp.dot(y_ref[...], Ga, preferred_element_type=jnp.float32)
    yB2 = jnp.dot(y_ref[...], Gb, preferred_element_type=jnp.float32)
    rb = c * yB1 * yB2 * den
    PG = P - Ga - Gb
    dyp_ref[...] += jax.lax.dot_general(
        rb, PG, (((1,), (1,)), ((), ())),
        preferred_element_type=jnp.float32)


def _corr_kernel(t_ref, y_ref, as_ref, bs_ref, gs_ref, ks_ref,
                 r1r_ref, r2r_ref, pr_ref, smtr_ref,
                 cnt_ref, ys_ref, ym_ref, dyp_ref, dy_ref):
    i = pl.program_id(0)

    @pl.when(i == 0)
    def _init():
        dy_ref[...] = dyp_ref[...]

    t = t_ref[...]
    nl = _LF * (ys_ref[...] + ym_ref[...])
    decay = jnp.minimum(_NAL / (nl + _EPS), 1.0)
    net = jnp.dot(dyp_ref[...], cnt_ref[...],
                  preferred_element_type=jnp.float32)
    sc = decay * _sigmoid(net)
    lsc = jnp.log(sc)
    L, invT = _med(t)
    ld = 4.0 + 2.0 * jnp.tanh(5e-4 * t)
    is2r = (smtr_ref[...] >= _R1).astype(jnp.float32)
    c = as_ref[...] * jnp.exp(bs_ref[...] * L - gs_ref[...] * invT
                              + is2r * ld)
    iota_sub = jax.lax.broadcasted_iota(jnp.int32, (_S, _CC), 0)
    G1 = (iota_sub == r1r_ref[...]).astype(jnp.float32)
    G2 = (iota_sub == r2r_ref[...]).astype(jnp.float32)
    yA = jnp.dot(y_ref[...], G1, preferred_element_type=jnp.float32)
    yB = jnp.dot(y_ref[...], G2, preferred_element_type=jnp.float32)
    yB = is2r * yB + (1.0 - is2r)
    k = ks_ref[...]
    scale = (jnp.exp(k * lsc) - 1.0) / k
    contrib = c * yA * yB * scale
    Pt = (iota_sub == pr_ref[...]).astype(jnp.float32)
    Mt = Pt - G1 - is2r * G2
    dy_ref[...] += jax.lax.dot_general(
        contrib, Mt, (((1,), (1,)), ((), ())),
        preferred_element_type=jnp.float32)


def _row(x, n):
    return x.astype(jnp.int32).reshape(1, n)


def _col(x, n):
    return x.astype(jnp.int32).reshape(n, 1)


def kernel(t_in, y_in, alpha_1st, beta_1st, gamma_1st, alpha_2nd, beta_2nd,
           gamma_2nd, r1_1st, p_1st, r1_2nd, r2_2nd, p_2nd,
           inds_surf, inds_mant, inds_smt):
    f32 = jnp.float32
    i32 = jnp.int32
    t_col = t_in.astype(f32).reshape(_B, 1)
    y = y_in.astype(f32)
    a1 = alpha_1st.astype(f32).reshape(1, _R1)
    b1 = beta_1st.astype(f32).reshape(1, _R1)
    g1 = gamma_1st.astype(f32).reshape(1, _R1)
    a2 = alpha_2nd.astype(f32).reshape(1, _R2)
    b2 = beta_2nd.astype(f32).reshape(1, _R2)
    g2 = gamma_2nd.astype(f32).reshape(1, _R2)

    const = lambda *bs: pl.BlockSpec(bs, lambda i: (0,) * len(bs))
    rowblk = lambda c: pl.BlockSpec((1, c), lambda i: (0, i))
    colblk = lambda c: pl.BlockSpec((c, 1), lambda i: (i, 0))

    n1 = _R1 // _CP
    ysurf, ymant, cnt, kmat, dyp1 = pl.pallas_call(
        _p1st_kernel,
        grid=(n1,),
        in_specs=[
            const(_B, 1), const(_B, _S),
            rowblk(_CP), rowblk(_CP), rowblk(_CP),
            rowblk(_CP), rowblk(_CP),
            const(1, _NS), const(1, _NM),
            const(1, _NSMT), const(_NSMT, 1),
        ],
        out_specs=[
            const(_B, 1), const(_B, 1),
            const(_S, 1), const(_HI, 128), const(_B, _S),
        ],
        out_shape=[
            jax.ShapeDtypeStruct((_B, 1), f32),
            jax.ShapeDtypeStruct((_B, 1), f32),
            jax.ShapeDtypeStruct((_S, 1), f32),
            jax.ShapeDtypeStruct((_HI, 128), f32),
            jax.ShapeDtypeStruct((_B, _S), f32),
        ],
    )(t_col, y, a1, b1, g1, _row(r1_1st, _R1), _row(p_1st, _R1),
      _row(inds_surf, _NS), _row(inds_mant, _NM),
      _row(inds_smt, _NSMT), _col(inds_smt, _NSMT))

    n2 = _R2 // _CP
    dyp = pl.pallas_call(
        _p2nd_kernel,
        grid=(n2,),
        in_specs=[
            const(_B, 1), const(_B, _S),
            rowblk(_CP), rowblk(_CP), rowblk(_CP),
            rowblk(_CP), rowblk(_CP), rowblk(_CP),
            const(_B, _S),
        ],
        out_specs=const(_B, _S),
        out_shape=jax.ShapeDtypeStruct((_B, _S), f32),
    )(t_col, y, a2, b2, g2, _row(r1_2nd, _R2), _row(r2_2nd, _R2),
      _row(p_2nd, _R2), dyp1)

    acat = jnp.concatenate([alpha_1st, alpha_2nd]).astype(f32)
    bcat = jnp.concatenate([beta_1st, beta_2nd]).astype(f32)
    gcat = jnp.concatenate([gamma_1st, gamma_2nd]).astype(f32)
    r1cat = jnp.concatenate([r1_1st, r1_2nd]).astype(i32)
    r2cat = jnp.concatenate([r1_1st, r2_2nd]).astype(i32)
    pcat = jnp.concatenate([p_1st, p_2nd]).astype(i32)
    kflat = kmat.reshape(_R)
    smt = inds_smt.astype(i32)

    a_s, b_s, g_s, k_s, r1_s, r2_s, p_s = _gather_params(
        acat, bcat, gcat, kflat, r1cat, r2cat, pcat, smt)

    nc = _NSMT // _CC
    dy = pl.pallas_call(
        _corr_kernel,
        grid=(nc,),
        in_specs=[
            const(_B, 1), const(_B, _S),
            rowblk(_CC), rowblk(_CC), rowblk(_CC), rowblk(_CC),
            rowblk(_CC), rowblk(_CC), rowblk(_CC), rowblk(_CC),
            const(_S, 1), const(_B, 1), const(_B, 1),
            const(_B, _S),
        ],
        out_specs=const(_B, _S),
        out_shape=jax.ShapeDtypeStruct((_B, _S), f32),
    )(t_col, y,
      a_s.reshape(1, _NSMT), b_s.reshape(1, _NSMT), g_s.reshape(1, _NSMT),
      k_s.reshape(1, _NSMT),
      _row(r1_s, _NSMT), _row(r2_s, _NSMT), _row(p_s, _NSMT),
      _row(smt, _NSMT),
      cnt, ysurf, ymant, dyp)

    return dy
